# Initial kernel scaffold; baseline (speedup 1.0000x reference)
import jax
import jax.numpy as jnp
from jax.experimental import pallas as pl


def kernel(x, edge_attr, edge_index, W, b):
    raise NotImplementedError("write your pallas kernel here")


# speedup vs baseline: 3.0216x; 3.0216x over previous
import functools

import jax
import jax.numpy as jnp
from jax import lax
from jax.experimental import pallas as pl
from jax.experimental.pallas import tpu as pltpu
from jax.experimental.pallas import tpu_sc as plsc

D = 128
NC, NS = 2, 16
NW = NC * NS
CHUNK = 128


def _node_tables_kernel(x_ref, w2_ref, w3_ref, t2_ref, t3_ref):
    xb = x_ref[...]
    t2_ref[...] = jnp.dot(xb, w2_ref[...], preferred_element_type=jnp.float32)
    t3_ref[...] = jnp.dot(xb, w3_ref[...], preferred_element_type=jnp.float32)


def _edge_out_kernel(ea_ref, g2_ref, g3_ref, w1_ref, b_ref, o_ref):
    o_ref[...] = (
        jnp.dot(ea_ref[...], w1_ref[...], preferred_element_type=jnp.float32)
        + g2_ref[...] + g3_ref[...] + b_ref[...]
    )


def _sc_gather(t2, t3, senders, receivers):
    n_edges = senders.shape[0]
    n_chunks = n_edges // CHUNK
    base_chunks = n_chunks // NW
    rem = n_chunks % NW

    mesh = plsc.VectorSubcoreMesh(core_axis_name="c", subcore_axis_name="s")

    @functools.partial(
        pl.kernel,
        out_type=[jax.ShapeDtypeStruct((n_edges, D), jnp.float32)] * 2,
        mesh=mesh,
        scratch_types=[
            pltpu.VMEM((CHUNK,), jnp.int32),
            pltpu.VMEM((CHUNK,), jnp.int32),
            pltpu.VMEM((CHUNK, D), jnp.float32),
            pltpu.VMEM((CHUNK, D), jnp.float32),
            pltpu.SemaphoreType.DMA,
        ],
    )
    def gather_k(t2_hbm, t3_hbm, s_hbm, r_hbm, g2_hbm, g3_hbm,
                 sidx_v, ridx_v, a_v, b_v, sem):
        wid = lax.axis_index("s") * NC + lax.axis_index("c")
        my_chunks = base_chunks + jnp.where(wid < rem, 1, 0)

        def body(k, carry):
            off = (wid + k * NW) * CHUNK
            pltpu.sync_copy(s_hbm.at[pl.ds(off, CHUNK)], sidx_v)
            pltpu.sync_copy(r_hbm.at[pl.ds(off, CHUNK)], ridx_v)
            c1 = pltpu.async_copy(t2_hbm.at[sidx_v], a_v, sem)
            c2 = pltpu.async_copy(t3_hbm.at[ridx_v], b_v, sem)
            c1.wait()
            c2.wait()
            pltpu.sync_copy(a_v, g2_hbm.at[pl.ds(off, CHUNK)])
            pltpu.sync_copy(b_v, g3_hbm.at[pl.ds(off, CHUNK)])
            return carry

        lax.fori_loop(0, my_chunks, body, 0)

    return gather_k(t2, t3, senders, receivers)


def kernel(x, edge_attr, edge_index, W, b):
    n_nodes, d = x.shape
    n_edges = edge_attr.shape[0]
    senders = edge_index[0].astype(jnp.int32)
    receivers = edge_index[1].astype(jnp.int32)
    W1, W2, W3 = W[:d], W[d:2 * d], W[2 * d:]

    nb = 5
    node_rows = n_nodes // nb
    t2, t3 = pl.pallas_call(
        _node_tables_kernel,
        grid=(nb,),
        in_specs=[
            pl.BlockSpec((node_rows, d), lambda i: (i, 0)),
            pl.BlockSpec((d, d), lambda i: (0, 0)),
            pl.BlockSpec((d, d), lambda i: (0, 0)),
        ],
        out_specs=[
            pl.BlockSpec((node_rows, d), lambda i: (i, 0)),
            pl.BlockSpec((node_rows, d), lambda i: (i, 0)),
        ],
        out_shape=[jax.ShapeDtypeStruct((n_nodes, d), jnp.float32)] * 2,
    )(x, W2, W3)

    g2, g3 = _sc_gather(t2, t3, senders, receivers)

    eb = 2560
    out = pl.pallas_call(
        _edge_out_kernel,
        grid=(n_edges // eb,),
        in_specs=[
            pl.BlockSpec((eb, d), lambda i: (i, 0)),
            pl.BlockSpec((eb, d), lambda i: (i, 0)),
            pl.BlockSpec((eb, d), lambda i: (i, 0)),
            pl.BlockSpec((d, d), lambda i: (0, 0)),
            pl.BlockSpec((1, d), lambda i: (0, 0)),
        ],
        out_specs=pl.BlockSpec((eb, d), lambda i: (i, 0)),
        out_shape=jax.ShapeDtypeStruct((n_edges, d), jnp.float32),
    )(edge_attr, g2, g3, W1, b.reshape(1, d))
    return out


# speedup vs baseline: 3.2863x; 1.0876x over previous
import functools

import jax
import jax.numpy as jnp
from jax import lax
from jax.experimental import pallas as pl
from jax.experimental.pallas import tpu as pltpu
from jax.experimental.pallas import tpu_sc as plsc

D = 128
NC, NS = 2, 16
NW = NC * NS
CHUNK = 128


def _node_tables_kernel(x_ref, w2_ref, w3_ref, t2_ref, t3_ref):
    xb = x_ref[...]
    t2_ref[...] = jnp.dot(xb, w2_ref[...], preferred_element_type=jnp.float32)
    t3_ref[...] = jnp.dot(xb, w3_ref[...], preferred_element_type=jnp.float32)


def _edge_out_kernel(ea_ref, g2_ref, g3_ref, w1_ref, b_ref, o_ref):
    o_ref[...] = (
        jnp.dot(ea_ref[...], w1_ref[...], preferred_element_type=jnp.float32)
        + g2_ref[...] + g3_ref[...] + b_ref[...]
    )


def _sc_gather(t2, t3, idx_all):
    n_idx = idx_all.shape[0]
    n_chunks_per_core = (n_idx // CHUNK) // NC
    base_chunks = n_chunks_per_core // NS
    rem = n_chunks_per_core % NS

    n_nodes = t2.shape[0]
    rows_per_sub = (n_nodes // NS) & ~7
    tail_rows = n_nodes - NS * rows_per_sub
    mesh = plsc.VectorSubcoreMesh(core_axis_name="c", subcore_axis_name="s")

    @functools.partial(
        pl.kernel,
        out_type=jax.ShapeDtypeStruct((n_idx, D), jnp.float32),
        mesh=mesh,
        scratch_types=[
            pltpu.VMEM((CHUNK,), jnp.int32),
            pltpu.VMEM((CHUNK, D), jnp.float32),
            pltpu.VMEM_SHARED((n_nodes, D), jnp.float32),
            pltpu.SemaphoreType.DMA,
        ],
    )
    def gather_k(t2_hbm, t3_hbm, idx_hbm, g_hbm, idx_v, a_v, t_sh, sem):
        cid = lax.axis_index("c")
        sid = lax.axis_index("s")

        roff = sid * rows_per_sub

        @pl.when(cid == 0)
        def _stage_t2():
            pltpu.sync_copy(t2_hbm.at[pl.ds(roff, rows_per_sub)],
                            t_sh.at[pl.ds(roff, rows_per_sub)])

        @pl.when(cid != 0)
        def _stage_t3():
            pltpu.sync_copy(t3_hbm.at[pl.ds(roff, rows_per_sub)],
                            t_sh.at[pl.ds(roff, rows_per_sub)])

        if tail_rows:
            toff = NS * rows_per_sub

            @pl.when((sid == 0) & (cid == 0))
            def _tail_t2():
                pltpu.sync_copy(t2_hbm.at[pl.ds(toff, tail_rows)],
                                t_sh.at[pl.ds(toff, tail_rows)])

            @pl.when((sid == 0) & (cid != 0))
            def _tail_t3():
                pltpu.sync_copy(t3_hbm.at[pl.ds(toff, tail_rows)],
                                t_sh.at[pl.ds(toff, tail_rows)])

        plsc.subcore_barrier()

        base_chunk = cid * n_chunks_per_core
        my_chunks = base_chunks + jnp.where(sid < rem, 1, 0)

        def body(k, carry):
            off = (base_chunk + sid + k * NS) * CHUNK
            pltpu.sync_copy(idx_hbm.at[pl.ds(off, CHUNK)], idx_v)
            pltpu.async_copy(t_sh.at[idx_v], a_v, sem).wait()
            pltpu.sync_copy(a_v, g_hbm.at[pl.ds(off, CHUNK)])
            return carry

        lax.fori_loop(0, my_chunks, body, 0)

    return gather_k(t2, t3, idx_all)


def kernel(x, edge_attr, edge_index, W, b):
    n_nodes, d = x.shape
    n_edges = edge_attr.shape[0]
    senders = edge_index[0].astype(jnp.int32)
    receivers = edge_index[1].astype(jnp.int32)
    W1, W2, W3 = W[:d], W[d:2 * d], W[2 * d:]

    nb = 5
    node_rows = n_nodes // nb
    t2, t3 = pl.pallas_call(
        _node_tables_kernel,
        grid=(nb,),
        in_specs=[
            pl.BlockSpec((node_rows, d), lambda i: (i, 0)),
            pl.BlockSpec((d, d), lambda i: (0, 0)),
            pl.BlockSpec((d, d), lambda i: (0, 0)),
        ],
        out_specs=[
            pl.BlockSpec((node_rows, d), lambda i: (i, 0)),
            pl.BlockSpec((node_rows, d), lambda i: (i, 0)),
        ],
        out_shape=[jax.ShapeDtypeStruct((n_nodes, d), jnp.float32)] * 2,
    )(x, W2, W3)

    idx_all = jnp.concatenate([senders, receivers])
    g = _sc_gather(t2, t3, idx_all)

    eb = 2560
    nblk = n_edges // eb
    out = pl.pallas_call(
        _edge_out_kernel,
        grid=(nblk,),
        in_specs=[
            pl.BlockSpec((eb, d), lambda i: (i, 0)),
            pl.BlockSpec((eb, d), lambda i: (i, 0)),
            pl.BlockSpec((eb, d), lambda i: (i + nblk, 0)),
            pl.BlockSpec((d, d), lambda i: (0, 0)),
            pl.BlockSpec((1, d), lambda i: (0, 0)),
        ],
        out_specs=pl.BlockSpec((eb, d), lambda i: (i, 0)),
        out_shape=jax.ShapeDtypeStruct((n_edges, d), jnp.float32),
    )(edge_attr, g, g, W1, b.reshape(1, d))
    return out


# speedup vs baseline: 4.0117x; 1.2207x over previous
import functools
import math

import jax
import jax.numpy as jnp
from jax import lax
from jax.experimental import pallas as pl
from jax.experimental.pallas import tpu as pltpu
from jax.experimental.pallas import tpu_sc as plsc

D = 128
NC, NS = 2, 16
CHUNK = 128
NBUF = 2
IDXBLK = 32


def _node_tables_kernel(x_ref, w2_ref, w3_ref, t2_ref, t3_ref):
    xb = x_ref[...]
    t2_ref[...] = jnp.dot(xb, w2_ref[...], preferred_element_type=jnp.float32)
    t3_ref[...] = jnp.dot(xb, w3_ref[...], preferred_element_type=jnp.float32)


def _edge_out_kernel(ea_ref, g2_ref, g3_ref, w1_ref, b_ref, o_ref):
    o_ref[...] = (
        jnp.dot(ea_ref[...], w1_ref[...], preferred_element_type=jnp.float32)
        + g2_ref[...] + g3_ref[...] + b_ref[...]
    )


def _sc_gather(t2, t3, idx2d):
    n_chunks, chunk = idx2d.shape
    assert chunk == CHUNK
    n_idx = n_chunks * chunk
    chunks_per_sub = n_chunks // (NC * NS)
    assert chunks_per_sub % IDXBLK == 0 and IDXBLK % NBUF == 0

    n_nodes = t2.shape[0]
    rows_per_sub = (n_nodes // NS) & ~7
    tail_rows = n_nodes - NS * rows_per_sub
    mesh = plsc.VectorSubcoreMesh(core_axis_name="c", subcore_axis_name="s")

    @functools.partial(
        pl.kernel,
        out_type=jax.ShapeDtypeStruct((n_idx, D), jnp.float32),
        mesh=mesh,
        scratch_types=[
            pltpu.VMEM((IDXBLK, CHUNK), jnp.int32),
            pltpu.VMEM((CHUNK, D), jnp.float32),
            pltpu.VMEM((CHUNK, D), jnp.float32),
            pltpu.VMEM_SHARED((n_nodes, D), jnp.float32),
            pltpu.SemaphoreType.DMA,
            pltpu.SemaphoreType.DMA,
            pltpu.SemaphoreType.DMA,
            pltpu.SemaphoreType.DMA,
        ],
    )
    def gather_k(t2_hbm, t3_hbm, idx_hbm, g_hbm,
                 idx_v, a0, a1, t_sh, sg0, sg1, sw0, sw1):
        cid = lax.axis_index("c")
        sid = lax.axis_index("s")

        roff = sid * rows_per_sub

        @pl.when(cid == 0)
        def _stage_t2():
            pltpu.sync_copy(t2_hbm.at[pl.ds(roff, rows_per_sub)],
                            t_sh.at[pl.ds(roff, rows_per_sub)])

        @pl.when(cid != 0)
        def _stage_t3():
            pltpu.sync_copy(t3_hbm.at[pl.ds(roff, rows_per_sub)],
                            t_sh.at[pl.ds(roff, rows_per_sub)])

        if tail_rows:
            toff = NS * rows_per_sub

            @pl.when((sid == 0) & (cid == 0))
            def _tail_t2():
                pltpu.sync_copy(t2_hbm.at[pl.ds(toff, tail_rows)],
                                t_sh.at[pl.ds(toff, tail_rows)])

            @pl.when((sid == 0) & (cid != 0))
            def _tail_t3():
                pltpu.sync_copy(t3_hbm.at[pl.ds(toff, tail_rows)],
                                t_sh.at[pl.ds(toff, tail_rows)])

        chunk0 = (cid * NS + sid) * chunks_per_sub
        plsc.subcore_barrier()

        slots = ((a0, sg0, sw0), (a1, sg1, sw1))
        steps_per_blk = IDXBLK // NBUF

        def blk_body(blk, carry):
            pltpu.sync_copy(
                idx_hbm.at[pl.ds(chunk0 + blk * IDXBLK, IDXBLK)], idx_v)

            def step(p, c2):
                for b, (buf, sg, sw) in enumerate(slots):
                    j = p * NBUF + b

                    @pl.when((blk > 0) | (p > 0))
                    def _wait_prev_write(buf=buf, sw=sw):
                        pltpu.make_async_copy(
                            g_hbm.at[pl.ds(0, CHUNK)], buf, sw).wait()

                    pltpu.async_copy(t_sh.at[idx_v.at[j]], buf, sg)
                for b, (buf, sg, sw) in enumerate(slots):
                    j = p * NBUF + b
                    pltpu.make_async_copy(
                        g_hbm.at[pl.ds(0, CHUNK)], buf, sg).wait()
                    off = (chunk0 + blk * IDXBLK + j) * CHUNK
                    pltpu.async_copy(buf, g_hbm.at[pl.ds(off, CHUNK)], sw)
                return c2

            lax.fori_loop(0, steps_per_blk, step, 0)
            return carry

        lax.fori_loop(0, chunks_per_sub // IDXBLK, blk_body, 0)
        for buf, _sg, sw in slots:
            pltpu.make_async_copy(g_hbm.at[pl.ds(0, CHUNK)], buf, sw).wait()

    return gather_k(t2, t3, idx2d)


def kernel(x, edge_attr, edge_index, W, b):
    n_nodes, d = x.shape
    n_edges = edge_attr.shape[0]
    senders = edge_index[0].astype(jnp.int32)
    receivers = edge_index[1].astype(jnp.int32)
    W1, W2, W3 = W[:d], W[d:2 * d], W[2 * d:]

    nb = 5
    node_rows = n_nodes // nb
    t2, t3 = pl.pallas_call(
        _node_tables_kernel,
        grid=(nb,),
        in_specs=[
            pl.BlockSpec((node_rows, d), lambda i: (i, 0)),
            pl.BlockSpec((d, d), lambda i: (0, 0)),
            pl.BlockSpec((d, d), lambda i: (0, 0)),
        ],
        out_specs=[
            pl.BlockSpec((node_rows, d), lambda i: (i, 0)),
            pl.BlockSpec((node_rows, d), lambda i: (i, 0)),
        ],
        out_shape=[jax.ShapeDtypeStruct((n_nodes, d), jnp.float32)] * 2,
    )(x, W2, W3)

    eb = 2560
    lcm = math.lcm(NS * CHUNK * IDXBLK, eb)
    half = -(-n_edges // lcm) * lcm
    pad = half - n_edges
    zpad = jnp.zeros((pad,), jnp.int32)
    idx_all = jnp.concatenate([senders, zpad, receivers, zpad])
    idx2d = idx_all.reshape(-1, CHUNK)

    g = _sc_gather(t2, t3, idx2d)

    nblk = n_edges // eb
    hblk = half // eb
    out = pl.pallas_call(
        _edge_out_kernel,
        grid=(nblk,),
        in_specs=[
            pl.BlockSpec((eb, d), lambda i: (i, 0)),
            pl.BlockSpec((eb, d), lambda i: (i, 0)),
            pl.BlockSpec((eb, d), lambda i: (i + hblk, 0)),
            pl.BlockSpec((d, d), lambda i: (0, 0)),
            pl.BlockSpec((1, d), lambda i: (0, 0)),
        ],
        out_specs=pl.BlockSpec((eb, d), lambda i: (i, 0)),
        out_shape=jax.ShapeDtypeStruct((n_edges, d), jnp.float32),
    )(edge_attr, g, g, W1, b.reshape(1, d))
    return out
